# baseline (device time: 115620 ns/iter reference)
import jax
import jax.numpy as jnp
from jax import lax
from jax.experimental import pallas as pl
from jax.experimental.pallas import tpu as pltpu

N_DEV = 8
B, S, H, Dh, Dr = 2, 512, 16, 128, 32
D = 2048
BS = B * S
HL = H // N_DEV
KC = HL * Dh
QRC = HL * Dr
KV = 2 * KC
SCALE = (Dh + Dr) ** -0.5
ARC = 3

f32 = jnp.float32
bf16 = jnp.bfloat16


def _body(x_ref, wdkv_ref, wuk_ref, wuv_ref, wq_ref, wqr_ref, wkr_ref,
          wo_hbm, out_ref,
          rs_snd_r, rs_snd_l, rs_rcv_r, rs_rcv_l, cd_snd, cd_rcv, o_gat,
          wo_ref, wo_stage,
          rs_ssem_r, rs_rsem_r, rs_ssem_l, rs_rsem_l,
          cd_ssem, cd_rsem,
          ag_ssem_r, ag_rsem_r, ag_ssem_l, ag_rsem_l,
          ag_cd_ssem, ag_cd_rsem, wo_sem):
    my = lax.axis_index("i")
    right = lax.rem(my + 1, N_DEV)
    left = lax.rem(my + N_DEV - 1, N_DEV)
    anti = lax.rem(my + 4, N_DEV)

    def pos(k):
        return lax.rem(my + k, N_DEV)

    def rdma(src, dst, ssem, rsem, dev):
        return pltpu.make_async_remote_copy(
            src_ref=src, dst_ref=dst, send_sem=ssem, recv_sem=rsem,
            device_id=(dev,), device_id_type=pl.DeviceIdType.MESH)

    QR = D // 4
    wo_dma = [pltpu.make_async_copy(wo_hbm.at[pl.ds(qd * QR, QR)],
                                    wo_stage.at[qd % 2], wo_sem.at[qd])
              for qd in range(4)]
    wo_dma[0].start()
    wo_dma[1].start()

    x = x_ref[...].astype(bf16)
    c = jnp.dot(x, wdkv_ref[...],
                preferred_element_type=f32).astype(bf16)

    def kv_put(dst, i, extra=None):
        k = jnp.dot(c, wuk_ref[:, pl.ds(i * KC, KC)],
                    preferred_element_type=f32)
        v = jnp.dot(c, wuv_ref[:, pl.ds(i * KC, KC)],
                    preferred_element_type=f32)
        if extra is not None:
            k = k + extra[:, :KC].astype(f32)
            v = v + extra[:, KC:].astype(f32)
        dst[:, :KC] = k.astype(bf16)
        dst[:, KC:] = v.astype(bf16)

    def kv_val(i):
        k = jnp.dot(c, wuk_ref[:, pl.ds(i * KC, KC)],
                    preferred_element_type=f32)
        v = jnp.dot(c, wuv_ref[:, pl.ds(i * KC, KC)],
                    preferred_element_type=f32)
        return k, v

    rs_r = [rdma(rs_snd_r, rs_rcv_r.at[s], rs_ssem_r.at[s],
                 rs_rsem_r.at[s], right) for s in range(ARC)]
    rs_l = [rdma(rs_snd_l, rs_rcv_l.at[s], rs_ssem_l.at[s],
                 rs_rsem_l.at[s], left) for s in range(ARC)]
    chord = rdma(cd_snd, cd_rcv, cd_ssem.at[0], cd_rsem.at[0], anti)

    kv_put(cd_snd, pos(4))
    chord.start()
    kv_put(rs_snd_r, pos(3))
    kv_put(rs_snd_l, pos(N_DEV - 3))
    rs_r[0].start()
    rs_l[0].start()

    q = jnp.dot(x, wq_ref[...],
                preferred_element_type=f32).astype(bf16)
    qr = jnp.dot(x, wqr_ref[...],
                 preferred_element_type=f32).astype(bf16)
    kr = jnp.dot(x, wkr_ref[...],
                 preferred_element_type=f32).astype(bf16)
    pk_r, pv_r = kv_val(pos(2))
    pk_l, pv_l = kv_val(pos(N_DEV - 2))

    for s in range(ARC - 1):
        rs_r[s].wait()
        rs_l[s].wait()
        rs_snd_r[:, :KC] = (rs_rcv_r[s, :, :KC].astype(f32) + pk_r).astype(bf16)
        rs_snd_r[:, KC:] = (rs_rcv_r[s, :, KC:].astype(f32) + pv_r).astype(bf16)
        rs_snd_l[:, :KC] = (rs_rcv_l[s, :, :KC].astype(f32) + pk_l).astype(bf16)
        rs_snd_l[:, KC:] = (rs_rcv_l[s, :, KC:].astype(f32) + pv_l).astype(bf16)
        rs_r[s + 1].start()
        rs_l[s + 1].start()
        if s < ARC - 2:
            pk_r, pv_r = kv_val(pos(1))
            pk_l, pv_l = kv_val(pos(N_DEV - 1))
    ok, ov = kv_val(pos(0))
    for qd in range(4):
        wo_dma[qd].wait()
        wo_ref[qd * QR:(qd + 1) * QR, :] = wo_stage[qd % 2].astype(bf16)
        if qd + 2 < 4:
            wo_dma[qd + 2].start()
    rs_r[ARC - 1].wait()
    rs_l[ARC - 1].wait()
    chord.wait()
    k_mine = (rs_rcv_r[ARC - 1, :, :KC].astype(f32)
              + rs_rcv_l[ARC - 1, :, :KC].astype(f32)
              + cd_rcv[:, :KC].astype(f32) + ok).astype(bf16)
    v_mine = (rs_rcv_r[ARC - 1, :, KC:].astype(f32)
              + rs_rcv_l[ARC - 1, :, KC:].astype(f32)
              + cd_rcv[:, KC:].astype(f32) + ov).astype(bf16)

    for b in range(B):
        rows = slice(b * S, (b + 1) * S)
        krb = kr[rows]
        for j in range(HL):
            cols = slice(j * Dh, (j + 1) * Dh)
            qh = q[rows, cols]
            kh = k_mine[rows, cols]
            qrh = qr[rows, j * Dr:(j + 1) * Dr]
            sc = lax.dot_general(qh, kh, (((1,), (1,)), ((), ())),
                                 preferred_element_type=f32)
            sc = sc + lax.dot_general(qrh, krb, (((1,), (1,)), ((), ())),
                                      preferred_element_type=f32)
            sc = sc * SCALE
            m = jnp.max(sc, axis=-1, keepdims=True)
            e = jnp.exp(sc - m)
            p = (e / jnp.sum(e, axis=-1, keepdims=True)).astype(bf16)
            o_gat[rows, pl.ds(my * KC + j * Dh, Dh)] = jnp.dot(
                p, v_mine[rows, cols], preferred_element_type=f32
            ).astype(bf16)

    def o_piece(i):
        return o_gat.at[:, pl.ds(i * KC, KC)]

    def wo_rows(i):
        return wo_ref[pl.ds(i * KC, KC), :]

    ag_cd = rdma(o_piece(my), o_piece(my), ag_cd_ssem.at[0],
                 ag_cd_rsem.at[0], anti)
    ag_cd.start()
    ag_r0 = rdma(o_piece(my), o_piece(my), ag_ssem_r.at[0],
                 ag_rsem_r.at[0], right)
    ag_l0 = rdma(o_piece(my), o_piece(my), ag_ssem_l.at[0],
                 ag_rsem_l.at[0], left)
    ag_r0.start()
    ag_l0.start()
    ag_r, ag_l = [ag_r0], [ag_l0]
    out_ref[...] = jnp.dot(o_gat[:, pl.ds(my * KC, KC)], wo_rows(my),
                           preferred_element_type=f32)
    for h in range(ARC):
        ag_r[h].wait()
        ag_l[h].wait()
        r_o = pos(2 * N_DEV - 1 - h)
        l_o = pos(1 + h)
        if h < ARC - 1:
            ag_r.append(rdma(o_piece(r_o), o_piece(r_o),
                             ag_ssem_r.at[h + 1], ag_rsem_r.at[h + 1], right))
            ag_l.append(rdma(o_piece(l_o), o_piece(l_o),
                             ag_ssem_l.at[h + 1], ag_rsem_l.at[h + 1], left))
            ag_r[h + 1].start()
            ag_l[h + 1].start()
        out_ref[...] = out_ref[...] + jnp.dot(
            o_gat[:, pl.ds(r_o * KC, KC)], wo_rows(r_o),
            preferred_element_type=f32)
        out_ref[...] = out_ref[...] + jnp.dot(
            o_gat[:, pl.ds(l_o * KC, KC)], wo_rows(l_o),
            preferred_element_type=f32)
    ag_cd.wait()
    a_o = pos(4)
    out_ref[...] = out_ref[...] + jnp.dot(
        o_gat[:, pl.ds(a_o * KC, KC)], wo_rows(a_o),
        preferred_element_type=f32)


def kernel(x, Wdkv, Wuk, Wuv, Wq, Wqr, Wkr, Wo):
    idx = lax.axis_index("i")
    xf = x.reshape(BS, D)
    wq_loc = lax.dynamic_slice(Wq, (0, idx * KC), (D, KC)).astype(bf16)
    wqr_loc = lax.dynamic_slice(Wqr, (0, idx * QRC), (D, QRC)).astype(bf16)

    out = pl.pallas_call(
        _body,
        out_shape=jax.ShapeDtypeStruct((BS, D), jnp.float32),
        in_specs=[pl.BlockSpec(memory_space=pltpu.VMEM)] * 7
        + [pl.BlockSpec(memory_space=pl.ANY)],
        out_specs=pl.BlockSpec(memory_space=pltpu.VMEM),
        scratch_shapes=[
            pltpu.VMEM((BS, KV), bf16),
            pltpu.VMEM((BS, KV), bf16),
            pltpu.VMEM((ARC, BS, KV), bf16),
            pltpu.VMEM((ARC, BS, KV), bf16),
            pltpu.VMEM((BS, KV), bf16),
            pltpu.VMEM((BS, KV), bf16),
            pltpu.VMEM((BS, D), bf16),
            pltpu.VMEM((D, D), bf16),
            pltpu.VMEM((2, D // 4, D), f32),
            pltpu.SemaphoreType.DMA((ARC,)),
            pltpu.SemaphoreType.DMA((ARC,)),
            pltpu.SemaphoreType.DMA((ARC,)),
            pltpu.SemaphoreType.DMA((ARC,)),
            pltpu.SemaphoreType.DMA((1,)),
            pltpu.SemaphoreType.DMA((1,)),
            pltpu.SemaphoreType.DMA((ARC,)),
            pltpu.SemaphoreType.DMA((ARC,)),
            pltpu.SemaphoreType.DMA((ARC,)),
            pltpu.SemaphoreType.DMA((ARC,)),
            pltpu.SemaphoreType.DMA((1,)),
            pltpu.SemaphoreType.DMA((1,)),
            pltpu.SemaphoreType.DMA((4,)),
        ],
        compiler_params=pltpu.CompilerParams(
            vmem_limit_bytes=62 * 1024 * 1024,
        ),
    )(xf, Wdkv.astype(bf16), Wuk.astype(bf16), Wuv.astype(bf16),
      wq_loc, wqr_loc, Wkr.astype(bf16), Wo)
    return out.reshape(B, S, D)
